# baseline (speedup 1.0000x reference)
import functools

import jax
import jax.numpy as jnp
from jax.experimental import pallas as pl
from jax.experimental.pallas import tpu as pltpu

_NPOINTS = [512, 256, 128, 64]
_RADII = [0.15, 0.3, 0.6, 1.2]
_K = 32


def _stem_kernel(f_ref, w_ref, g_ref, b_ref, o_ref):
    x = jnp.dot(f_ref[0], w_ref[...], preferred_element_type=jnp.float32)
    o_ref[0] = jnp.maximum(x * g_ref[...] + b_ref[...], 0.0)


def _fps_kernel(xt_ref, idx_ref, *, S):
    xt = xt_ref[...]
    x = xt[:, 0, :]
    y = xt[:, 1, :]
    z = xt[:, 2, :]
    B, N = x.shape
    iN = jax.lax.broadcasted_iota(jnp.int32, (B, N), 1)
    iS = jax.lax.broadcasted_iota(jnp.int32, (B, S), 1)

    idx_ref[...] = jnp.zeros((B, S), jnp.int32)

    def body(i, st):
        dists, far = st
        sel = iN == far
        cx = jnp.sum(jnp.where(sel, x, 0.0), axis=1, keepdims=True)
        cy = jnp.sum(jnp.where(sel, y, 0.0), axis=1, keepdims=True)
        cz = jnp.sum(jnp.where(sel, z, 0.0), axis=1, keepdims=True)
        idx_ref[...] = jnp.where(iS == i, far, idx_ref[...])
        d = (x - cx) ** 2 + (y - cy) ** 2 + (z - cz) ** 2
        dists = jnp.minimum(dists, d)
        m = jnp.max(dists, axis=1, keepdims=True)
        far = jnp.min(jnp.where(dists == m, iN, N), axis=1, keepdims=True)
        far = far.astype(jnp.int32)
        return (dists, far)

    init = (
        jnp.full((B, N), 1e10, jnp.float32) + 0.0 * x,
        jnp.zeros((B, 1), jnp.int32),
    )
    jax.lax.fori_loop(0, S, body, init)


def _stage_kernel(xyz_ref, xyzT_ref, feats_ref, fidx_ref,
                  w0_ref, g0_ref, b0_ref, w1_ref, g1_ref, b1_ref, wres_ref,
                  o_ref, oxyz_ref, *, K, radius):
    xyz = xyz_ref[0]
    xyzT = xyzT_ref[0]
    feats = feats_ref[0]
    fidx = fidx_ref[0, 0]
    N = xyz.shape[0]
    Sb = fidx.shape[0]

    pts = jnp.concatenate([xyz, feats], axis=1)
    iN = jax.lax.broadcasted_iota(jnp.int32, (1, N), 1)
    sel = fidx == iN
    nxc = []
    for d in range(3):
        nxc.append(jnp.sum(jnp.where(sel, xyzT[d:d + 1, :], 0.0),
                           axis=1, keepdims=True))
    nxyz = jnp.concatenate(nxc, axis=1)
    ohfps = sel.astype(jnp.float32)
    nall = jnp.dot(ohfps, pts, preferred_element_type=jnp.float32)
    oxyz_ref[0] = nxyz

    dx = nxyz[:, 0:1] - xyzT[0:1, :]
    dy = nxyz[:, 1:2] - xyzT[1:2, :]
    dz = nxyz[:, 2:3] - xyzT[2:3, :]
    dist2 = dx * dx + dy * dy + dz * dz
    mask = dist2 <= radius * radius
    maskf = mask.astype(jnp.float32)
    rank = maskf
    shift = 1
    while shift < N:
        z = jnp.zeros((Sb, shift), jnp.float32)
        rank = rank + jnp.concatenate([z, rank[:, :N - shift]], axis=1)
        shift *= 2

    ranki = rank.astype(jnp.int32)[None, :, :]
    mask3 = mask[None, :, :]
    count2 = jnp.sum(maskf, axis=1, keepdims=True).astype(jnp.int32)
    count3 = count2[None, :, :]
    kio = jax.lax.broadcasted_iota(jnp.int32, (K, Sb, N), 0)
    target = jnp.minimum(kio + 1, count3)
    oh = (ranki == target) & mask3
    ohf = oh.astype(jnp.float32).reshape(K * Sb, N)

    g = jnp.dot(ohf, pts, preferred_element_type=jnp.float32)
    rep = jnp.broadcast_to(nxyz[None, :, :], (K, Sb, 3)).reshape(K * Sb, 3)
    dp = (g[:, 0:3] - rep) * (1.0 / radius)
    hcat = jnp.concatenate([dp, g[:, 3:]], axis=1)

    h = jnp.dot(hcat, w0_ref[...], preferred_element_type=jnp.float32)
    h = jnp.maximum(h * g0_ref[...] + b0_ref[...], 0.0)
    h = jnp.dot(h, w1_ref[...], preferred_element_type=jnp.float32)
    h = jnp.maximum(h * g1_ref[...] + b1_ref[...], 0.0)
    cout = h.shape[1]
    h3 = h.reshape(K, Sb, cout)
    pooled = jnp.max(h3, axis=0)

    res = jnp.dot(nall[:, 3:], wres_ref[...], preferred_element_type=jnp.float32)
    o_ref[0] = jnp.maximum(pooled + res, 0.0)


def _head_kernel(f_ref, w1_ref, b1_ref, g_ref, bb_ref, w2_ref, b2_ref, o_ref):
    gmax = jnp.max(f_ref[...], axis=1)
    h = jnp.dot(gmax, w1_ref[...], preferred_element_type=jnp.float32)
    h = h + b1_ref[...]
    h = jnp.maximum(h * g_ref[...] + bb_ref[...], 0.0)
    o_ref[...] = jnp.dot(h, w2_ref[...], preferred_element_type=jnp.float32) + b2_ref[...]


def _stem(features, params):
    B, N, _ = features.shape
    W = params['stem_W']
    C = W.shape[1]
    return pl.pallas_call(
        _stem_kernel,
        grid=(B,),
        in_specs=[
            pl.BlockSpec((1, N, 3), lambda b: (b, 0, 0)),
            pl.BlockSpec((3, C), lambda b: (0, 0)),
            pl.BlockSpec((1, C), lambda b: (0, 0)),
            pl.BlockSpec((1, C), lambda b: (0, 0)),
        ],
        out_specs=pl.BlockSpec((1, N, C), lambda b: (b, 0, 0)),
        out_shape=jax.ShapeDtypeStruct((B, N, C), jnp.float32),
        compiler_params=pltpu.CompilerParams(
            dimension_semantics=("parallel",)),
    )(features, W, params['stem_g'][None], params['stem_b'][None])


def _fps(xyzT, S):
    B, _, N = xyzT.shape
    fn = functools.partial(_fps_kernel, S=S)
    return pl.pallas_call(
        fn,
        out_shape=jax.ShapeDtypeStruct((B, S), jnp.int32),
    )(xyzT)


def _stage(cur_xyz, xyzT, cur_feats, fidx, params, i, radius):
    B, N, _ = cur_xyz.shape
    C = cur_feats.shape[2]
    S = fidx.shape[1]
    W0 = params['s%d_W0' % i]
    W1 = params['s%d_W1' % i]
    Wres = params['s%d_Wres' % i]
    cin, cout = W0.shape
    Sb = min(256, S)
    T = S // Sb
    fidx_r = fidx.reshape(B, T, Sb, 1)
    fn = functools.partial(_stage_kernel, K=_K, radius=radius)
    return pl.pallas_call(
        fn,
        grid=(B, T),
        in_specs=[
            pl.BlockSpec((1, N, 3), lambda b, s: (b, 0, 0)),
            pl.BlockSpec((1, 3, N), lambda b, s: (b, 0, 0)),
            pl.BlockSpec((1, N, C), lambda b, s: (b, 0, 0)),
            pl.BlockSpec((1, 1, Sb, 1), lambda b, s: (b, s, 0, 0)),
            pl.BlockSpec((cin, cout), lambda b, s: (0, 0)),
            pl.BlockSpec((1, cout), lambda b, s: (0, 0)),
            pl.BlockSpec((1, cout), lambda b, s: (0, 0)),
            pl.BlockSpec((cout, cout), lambda b, s: (0, 0)),
            pl.BlockSpec((1, cout), lambda b, s: (0, 0)),
            pl.BlockSpec((1, cout), lambda b, s: (0, 0)),
            pl.BlockSpec((C, cout), lambda b, s: (0, 0)),
        ],
        out_specs=(
            pl.BlockSpec((1, Sb, cout), lambda b, s: (b, s, 0)),
            pl.BlockSpec((1, Sb, 3), lambda b, s: (b, s, 0)),
        ),
        out_shape=(
            jax.ShapeDtypeStruct((B, S, cout), jnp.float32),
            jax.ShapeDtypeStruct((B, S, 3), jnp.float32),
        ),
        compiler_params=pltpu.CompilerParams(
            dimension_semantics=("parallel", "arbitrary")),
    )(cur_xyz, xyzT, cur_feats, fidx_r,
      W0, params['s%d_g0' % i][None], params['s%d_b0' % i][None],
      W1, params['s%d_g1' % i][None], params['s%d_b1' % i][None],
      Wres)


def _head(feats, params):
    B, S, C = feats.shape
    H = params['head_W1'].shape[1]
    NC = params['head_W2'].shape[1]
    return pl.pallas_call(
        _head_kernel,
        out_shape=jax.ShapeDtypeStruct((B, NC), jnp.float32),
    )(feats, params['head_W1'], params['head_b1'][None],
      params['head_g'][None], params['head_bb'][None],
      params['head_W2'], params['head_b2'][None])


def kernel(xyz, features, params):
    cur_feats = _stem(features, params)
    cur_xyz = xyz
    for i in range(4):
        S = _NPOINTS[i]
        xyzT = jnp.transpose(cur_xyz, (0, 2, 1))
        fidx = _fps(xyzT, S)
        cur_feats, cur_xyz = _stage(cur_xyz, xyzT, cur_feats, fidx,
                                    params, i, _RADII[i])
    return _head(cur_feats, params)


# speedup vs baseline: 1.1654x; 1.1654x over previous
import functools

import jax
import jax.numpy as jnp
from jax.experimental import pallas as pl
from jax.experimental.pallas import tpu as pltpu

_NPOINTS = [512, 256, 128, 64]
_RADII = [0.15, 0.3, 0.6, 1.2]
_K = 32


def _stem_kernel(f_ref, w_ref, g_ref, b_ref, o_ref):
    x = jnp.dot(f_ref[0], w_ref[...], preferred_element_type=jnp.float32)
    o_ref[0] = jnp.maximum(x * g_ref[...] + b_ref[...], 0.0)


def _fps_kernel(xc_ref, idx_ref, *, S, B):
    xcat = xc_ref[...]
    N = xcat.shape[1]
    x = xcat[0:B]
    y = xcat[B:2 * B]
    z = xcat[2 * B:3 * B]
    iN = jax.lax.broadcasted_iota(jnp.int32, (B, N), 1)
    iS = jax.lax.broadcasted_iota(jnp.int32, (B, S), 1)

    idx_ref[...] = jnp.zeros((B, S), jnp.int32)

    def body(i, st):
        dists, far = st
        sel = iN == far
        selcat = jnp.concatenate([sel, sel, sel], axis=0)
        c = jnp.sum(jnp.where(selcat, xcat, 0.0), axis=1, keepdims=True)
        cx = c[0:B]
        cy = c[B:2 * B]
        cz = c[2 * B:3 * B]
        idx_ref[...] = jnp.where(iS == i, far, idx_ref[...])
        d = (x - cx) ** 2 + (y - cy) ** 2 + (z - cz) ** 2
        dists = jnp.minimum(dists, d)
        m = jnp.max(dists, axis=1, keepdims=True)
        far = jnp.min(jnp.where(dists == m, iN, N), axis=1, keepdims=True)
        far = far.astype(jnp.int32)
        return (dists, far)

    init = (
        jnp.full((B, N), 1e10, jnp.float32) + 0.0 * x,
        jnp.zeros((B, 1), jnp.int32),
    )
    jax.lax.fori_loop(0, S, body, init)


def _stage_kernel(xyz_ref, xyzT_ref, feats_ref, fidx_ref,
                  w0_ref, g0_ref, b0_ref, w1_ref, g1_ref, b1_ref, wres_ref,
                  o_ref, oxyz_ref, *, K, radius):
    xyz = xyz_ref[0]
    xyzT = xyzT_ref[0]
    feats = feats_ref[0]
    fidx = fidx_ref[0, 0]
    N = xyz.shape[0]
    Sb = fidx.shape[0]

    pts = jnp.concatenate([xyz, feats], axis=1)
    iN = jax.lax.broadcasted_iota(jnp.int32, (1, N), 1)
    sel = fidx == iN
    nxc = []
    for d in range(3):
        nxc.append(jnp.sum(jnp.where(sel, xyzT[d:d + 1, :], 0.0),
                           axis=1, keepdims=True))
    nxyz = jnp.concatenate(nxc, axis=1)
    ohfps = sel.astype(jnp.float32)
    nall = jnp.dot(ohfps, pts, preferred_element_type=jnp.float32)
    oxyz_ref[0] = nxyz

    dx = nxyz[:, 0:1] - xyzT[0:1, :]
    dy = nxyz[:, 1:2] - xyzT[1:2, :]
    dz = nxyz[:, 2:3] - xyzT[2:3, :]
    dist2 = dx * dx + dy * dy + dz * dz
    mask = dist2 <= radius * radius
    maskf = mask.astype(jnp.float32)
    rank = maskf
    shift = 1
    while shift < N:
        z = jnp.zeros((Sb, shift), jnp.float32)
        rank = rank + jnp.concatenate([z, rank[:, :N - shift]], axis=1)
        shift *= 2

    ranki = rank.astype(jnp.int32)[None, :, :]
    mask3 = mask[None, :, :]
    count2 = jnp.sum(maskf, axis=1, keepdims=True).astype(jnp.int32)
    count3 = count2[None, :, :]
    kio = jax.lax.broadcasted_iota(jnp.int32, (K, Sb, N), 0)
    target = jnp.minimum(kio + 1, count3)
    oh = (ranki == target) & mask3
    ohf = oh.astype(jnp.float32).reshape(K * Sb, N)

    g = jnp.dot(ohf, pts, preferred_element_type=jnp.float32)
    rep = jnp.broadcast_to(nxyz[None, :, :], (K, Sb, 3)).reshape(K * Sb, 3)
    dp = (g[:, 0:3] - rep) * (1.0 / radius)
    hcat = jnp.concatenate([dp, g[:, 3:]], axis=1)

    h = jnp.dot(hcat, w0_ref[...], preferred_element_type=jnp.float32)
    h = jnp.maximum(h * g0_ref[...] + b0_ref[...], 0.0)
    h = jnp.dot(h, w1_ref[...], preferred_element_type=jnp.float32)
    h = jnp.maximum(h * g1_ref[...] + b1_ref[...], 0.0)
    cout = h.shape[1]
    h3 = h.reshape(K, Sb, cout)
    pooled = jnp.max(h3, axis=0)

    res = jnp.dot(nall[:, 3:], wres_ref[...], preferred_element_type=jnp.float32)
    o_ref[0] = jnp.maximum(pooled + res, 0.0)


def _head_kernel(f_ref, w1_ref, b1_ref, g_ref, bb_ref, w2_ref, b2_ref, o_ref):
    gmax = jnp.max(f_ref[...], axis=1)
    h = jnp.dot(gmax, w1_ref[...], preferred_element_type=jnp.float32)
    h = h + b1_ref[...]
    h = jnp.maximum(h * g_ref[...] + bb_ref[...], 0.0)
    o_ref[...] = jnp.dot(h, w2_ref[...], preferred_element_type=jnp.float32) + b2_ref[...]


def _stem(features, params):
    B, N, _ = features.shape
    W = params['stem_W']
    C = W.shape[1]
    return pl.pallas_call(
        _stem_kernel,
        grid=(B,),
        in_specs=[
            pl.BlockSpec((1, N, 3), lambda b: (b, 0, 0)),
            pl.BlockSpec((3, C), lambda b: (0, 0)),
            pl.BlockSpec((1, C), lambda b: (0, 0)),
            pl.BlockSpec((1, C), lambda b: (0, 0)),
        ],
        out_specs=pl.BlockSpec((1, N, C), lambda b: (b, 0, 0)),
        out_shape=jax.ShapeDtypeStruct((B, N, C), jnp.float32),
        compiler_params=pltpu.CompilerParams(
            dimension_semantics=("parallel",)),
    )(features, W, params['stem_g'][None], params['stem_b'][None])


def _fps(xcat, S, B):
    fn = functools.partial(_fps_kernel, S=S, B=B)
    return pl.pallas_call(
        fn,
        out_shape=jax.ShapeDtypeStruct((B, S), jnp.int32),
    )(xcat)


def _stage(cur_xyz, xyzT, cur_feats, fidx, params, i, radius):
    B, N, _ = cur_xyz.shape
    C = cur_feats.shape[2]
    S = fidx.shape[1]
    W0 = params['s%d_W0' % i]
    W1 = params['s%d_W1' % i]
    Wres = params['s%d_Wres' % i]
    cin, cout = W0.shape
    Sb = min(128, S)
    T = S // Sb
    fidx_r = fidx.reshape(B, T, Sb, 1)
    fn = functools.partial(_stage_kernel, K=_K, radius=radius)
    return pl.pallas_call(
        fn,
        grid=(B, T),
        in_specs=[
            pl.BlockSpec((1, N, 3), lambda b, s: (b, 0, 0)),
            pl.BlockSpec((1, 3, N), lambda b, s: (b, 0, 0)),
            pl.BlockSpec((1, N, C), lambda b, s: (b, 0, 0)),
            pl.BlockSpec((1, 1, Sb, 1), lambda b, s: (b, s, 0, 0)),
            pl.BlockSpec((cin, cout), lambda b, s: (0, 0)),
            pl.BlockSpec((1, cout), lambda b, s: (0, 0)),
            pl.BlockSpec((1, cout), lambda b, s: (0, 0)),
            pl.BlockSpec((cout, cout), lambda b, s: (0, 0)),
            pl.BlockSpec((1, cout), lambda b, s: (0, 0)),
            pl.BlockSpec((1, cout), lambda b, s: (0, 0)),
            pl.BlockSpec((C, cout), lambda b, s: (0, 0)),
        ],
        out_specs=(
            pl.BlockSpec((1, Sb, cout), lambda b, s: (b, s, 0)),
            pl.BlockSpec((1, Sb, 3), lambda b, s: (b, s, 0)),
        ),
        out_shape=(
            jax.ShapeDtypeStruct((B, S, cout), jnp.float32),
            jax.ShapeDtypeStruct((B, S, 3), jnp.float32),
        ),
        compiler_params=pltpu.CompilerParams(
            dimension_semantics=("parallel", "arbitrary")),
    )(cur_xyz, xyzT, cur_feats, fidx_r,
      W0, params['s%d_g0' % i][None], params['s%d_b0' % i][None],
      W1, params['s%d_g1' % i][None], params['s%d_b1' % i][None],
      Wres)


def _head(feats, params):
    B, S, C = feats.shape
    H = params['head_W1'].shape[1]
    NC = params['head_W2'].shape[1]
    return pl.pallas_call(
        _head_kernel,
        out_shape=jax.ShapeDtypeStruct((B, NC), jnp.float32),
    )(feats, params['head_W1'], params['head_b1'][None],
      params['head_g'][None], params['head_bb'][None],
      params['head_W2'], params['head_b2'][None])


def kernel(xyz, features, params):
    cur_feats = _stem(features, params)
    cur_xyz = xyz
    for i in range(4):
        S = _NPOINTS[i]
        B, N, _ = cur_xyz.shape
        xyzT = jnp.transpose(cur_xyz, (0, 2, 1))
        xcat = jnp.transpose(cur_xyz, (2, 0, 1)).reshape(3 * B, N)
        fidx = _fps(xcat, S, B)
        cur_feats, cur_xyz = _stage(cur_xyz, xyzT, cur_feats, fidx,
                                    params, i, _RADII[i])
    return _head(cur_feats, params)
